# baseline (speedup 1.0000x reference)
import functools

import jax
import jax.numpy as jnp
from jax import lax
from jax.experimental import pallas as pl
from jax.experimental.pallas import tpu as pltpu
from jax.experimental.pallas import tpu_sc as plsc

_B, _TB, _H, _W = 16, 512, 64, 64
_C, _K = 64, 4
_PLANE = _H * _W
_NW = 32
_PPW = (_B * _C) // _NW
_PPC = 2
_NCHUNK = _PPW // _PPC
_THRESH = 2.0


def _threshold_chunk(rows_ref, out_ref):
    def body(j, _):
        off = j * 16
        for p in range(_PPC):
            r0 = rows_ref[4 * p + 0, pl.ds(off, 16)]
            r1 = rows_ref[4 * p + 1, pl.ds(off, 16)]
            r2 = rows_ref[4 * p + 2, pl.ds(off, 16)]
            r3 = rows_ref[4 * p + 3, pl.ds(off, 16)]
            s = ((r0 + r1) + r2) + r3
            out_ref[p, pl.ds(off, 16)] = jnp.where(
                s >= _THRESH, jnp.float32(1.0), jnp.float32(0.0))
        return 0
    lax.fori_loop(0, _PLANE // 16, body, 0)


def _sc_body(x_hbm, idx_hbm, out_hbm,
             idx_v, rows_a, rows_b, out_a, out_b,
             gsem_a, gsem_b, osem_a, osem_b):
    wid = lax.axis_index("s") * 2 + lax.axis_index("c")
    pltpu.sync_copy(idx_hbm.at[wid], idx_v)

    rows = [rows_a, rows_b]
    outs = [out_a, out_b]
    gsems = [gsem_a, gsem_b]
    osems = [osem_a, osem_b]
    ghandles = [None, None]
    ohandles = [None, None]

    ghandles[0] = pltpu.async_copy(x_hbm.at[idx_v.at[0]], rows[0], gsems[0])
    for t in range(_NCHUNK):
        cur = t & 1
        nxt = 1 - cur
        if t + 1 < _NCHUNK:
            ghandles[nxt] = pltpu.async_copy(
                x_hbm.at[idx_v.at[t + 1]], rows[nxt], gsems[nxt])
        ghandles[cur].wait()
        if ohandles[cur] is not None:
            ohandles[cur].wait()
        _threshold_chunk(rows[cur], outs[cur])
        ohandles[cur] = pltpu.async_copy(
            outs[cur], out_hbm.at[pl.ds(wid * _PPW + _PPC * t, _PPC)],
            osems[cur])
    ohandles[0].wait()
    ohandles[1].wait()


_sc_kernel = functools.partial(
    pl.kernel,
    out_type=jax.ShapeDtypeStruct((_B * _C, _PLANE), jnp.float32),
    mesh=plsc.VectorSubcoreMesh(core_axis_name="c", subcore_axis_name="s"),
    scratch_types=[
        pltpu.VMEM((_NCHUNK, 8), jnp.int32),
        pltpu.VMEM((4 * _PPC, _PLANE), jnp.float32),
        pltpu.VMEM((4 * _PPC, _PLANE), jnp.float32),
        pltpu.VMEM((_PPC, _PLANE), jnp.float32),
        pltpu.VMEM((_PPC, _PLANE), jnp.float32),
        pltpu.SemaphoreType.DMA,
        pltpu.SemaphoreType.DMA,
        pltpu.SemaphoreType.DMA,
        pltpu.SemaphoreType.DMA,
    ],
)(_sc_body)


def kernel(x, rp_map_idx):
    xf = x.reshape(_B * _TB, _PLANE)
    base = (jnp.arange(_B, dtype=jnp.int32) * _TB)[:, None, None]
    flat = (base + rp_map_idx[None, :, :].astype(jnp.int32))
    idx = flat.reshape(_NW, _NCHUNK, _PPC * _K)
    y = _sc_kernel(xf, idx)
    return y.reshape(_B, _C, _H, _W)
